# baseline (speedup 1.0000x reference)
import jax
import jax.numpy as jnp
from jax.experimental import pallas as pl

C = 192
L = 20
H = 224
W = 224
P = H * W
BLK = 3584


def _fold_body(w_ref, b_ref, ahi_ref, alo_ref, c_ref):
    cols = jax.lax.broadcasted_iota(jnp.int32, (C, 256), 1)
    w0 = w_ref[0].astype(jnp.bfloat16).astype(jnp.float32)
    aug = jnp.concatenate(
        [w0, b_ref[0][:, None], jnp.zeros((C, 63), jnp.float32)], axis=1
    )
    for l in range(1, L):
        wlb = w_ref[l].astype(jnp.bfloat16)
        hi = aug.astype(jnp.bfloat16)
        lo = (aug - hi.astype(jnp.float32)).astype(jnp.bfloat16)
        aug = jnp.dot(wlb, hi, preferred_element_type=jnp.float32) + jnp.dot(
            wlb, lo, preferred_element_type=jnp.float32
        )
        aug = aug + jnp.where(cols == C, b_ref[l][:, None], 0.0)
    a = aug[:, :C]
    ahi = a.astype(jnp.bfloat16)
    ahi_ref[...] = ahi
    alo_ref[...] = (a - ahi.astype(jnp.float32)).astype(jnp.bfloat16)
    c_ref[...] = aug[:, C : C + 1]


def _apply_body(x_ref, ahi_ref, alo_ref, c_ref, o_ref):
    xt = x_ref[...].astype(jnp.bfloat16).reshape(C, BLK)
    acc = (
        jnp.dot(ahi_ref[...], xt, preferred_element_type=jnp.float32)
        + jnp.dot(alo_ref[...], xt, preferred_element_type=jnp.float32)
        + c_ref[...]
    )
    o_ref[...] = acc.reshape(1, C, BLK // W, W)


def kernel(x, W0, b0, W1, b1):
    ahi, alo, c = pl.pallas_call(
        _fold_body,
        out_shape=(
            jax.ShapeDtypeStruct((C, C), jnp.bfloat16),
            jax.ShapeDtypeStruct((C, C), jnp.bfloat16),
            jax.ShapeDtypeStruct((C, 1), jnp.float32),
        ),
    )(W0, b0)
    out = pl.pallas_call(
        _apply_body,
        grid=(P // BLK,),
        in_specs=[
            pl.BlockSpec((1, C, BLK // W, W), lambda i: (0, 0, i, 0)),
            pl.BlockSpec((C, C), lambda i: (0, 0)),
            pl.BlockSpec((C, C), lambda i: (0, 0)),
            pl.BlockSpec((C, 1), lambda i: (0, 0)),
        ],
        out_specs=pl.BlockSpec((1, C, BLK // W, W), lambda i: (0, 0, i, 0)),
        out_shape=jax.ShapeDtypeStruct((1, C, H, W), jnp.float32),
    )(x, ahi, alo, c)
    return out


# speedup vs baseline: 1.1031x; 1.1031x over previous
import jax
import jax.numpy as jnp
from jax.experimental import pallas as pl

C = 192
L = 20
H = 224
W = 224
P = H * W
BLK = 7168


def _dot(p, q):
    return jnp.dot(p, q, preferred_element_type=jnp.float32)


def _split(m):
    hi = m.astype(jnp.bfloat16)
    return hi, (m - hi.astype(jnp.float32)).astype(jnp.bfloat16)


def _fold_body(w_ref, b_ref, ahi_ref, alo_ref, c_ref):
    wb = [w_ref[l].astype(jnp.bfloat16) for l in range(L)]
    nodes = []
    for k in range(L // 2):
        a = _dot(wb[2 * k + 1], wb[2 * k])
        c = _dot(wb[2 * k + 1], b_ref[2 * k][:, None]) + b_ref[2 * k + 1][:, None]
        nodes.append((a, c))
    while len(nodes) > 1:
        nxt = []
        for k in range(0, len(nodes) - 1, 2):
            (al, cl), (ar, cr) = nodes[k], nodes[k + 1]
            alh, all_ = _split(al)
            arh, arl = _split(ar)
            a = _dot(arh, alh) + _dot(arh, all_) + _dot(arl, alh)
            c = _dot(arh, cl.astype(jnp.bfloat16)) + cr
            nxt.append((a, c))
        if len(nodes) % 2:
            nxt.append(nodes[-1])
        nodes = nxt
    a, c = nodes[0]
    ahi = a.astype(jnp.bfloat16)
    ahi_ref[...] = ahi
    alo_ref[...] = (a - ahi.astype(jnp.float32)).astype(jnp.bfloat16)
    c_ref[...] = c


def _apply_body(x_ref, ahi_ref, alo_ref, c_ref, o_ref):
    xt = x_ref[...].astype(jnp.bfloat16).reshape(C, BLK)
    acc = (
        jnp.dot(ahi_ref[...], xt, preferred_element_type=jnp.float32)
        + jnp.dot(alo_ref[...], xt, preferred_element_type=jnp.float32)
        + c_ref[...]
    )
    o_ref[...] = acc.reshape(1, C, BLK // W, W)


def kernel(x, W0, b0, W1, b1):
    ahi, alo, c = pl.pallas_call(
        _fold_body,
        out_shape=(
            jax.ShapeDtypeStruct((C, C), jnp.bfloat16),
            jax.ShapeDtypeStruct((C, C), jnp.bfloat16),
            jax.ShapeDtypeStruct((C, 1), jnp.float32),
        ),
    )(W0, b0)
    out = pl.pallas_call(
        _apply_body,
        grid=(P // BLK,),
        in_specs=[
            pl.BlockSpec((1, C, BLK // W, W), lambda i: (0, 0, i, 0)),
            pl.BlockSpec((C, C), lambda i: (0, 0)),
            pl.BlockSpec((C, C), lambda i: (0, 0)),
            pl.BlockSpec((C, 1), lambda i: (0, 0)),
        ],
        out_specs=pl.BlockSpec((1, C, BLK // W, W), lambda i: (0, 0, i, 0)),
        out_shape=jax.ShapeDtypeStruct((1, C, H, W), jnp.float32),
    )(x, ahi, alo, c)
    return out


# speedup vs baseline: 1.1480x; 1.0407x over previous
import jax
import jax.numpy as jnp
from jax.experimental import pallas as pl
from jax.experimental.pallas import tpu as pltpu

C = 192
L = 20
H = 224
W = 224
P = H * W
BLK = 7168


def _dot(p, q):
    return jnp.dot(p, q, preferred_element_type=jnp.float32)


def _split(m):
    hi = m.astype(jnp.bfloat16)
    return hi, (m - hi.astype(jnp.float32)).astype(jnp.bfloat16)


def _fused_body(x_ref, w_ref, b_ref, o_ref, ahi_ref, alo_ref, c_ref):
    @pl.when(pl.program_id(0) == 0)
    def _():
        _fold_compute(w_ref, b_ref, ahi_ref, alo_ref, c_ref)

    xt = x_ref[...].astype(jnp.bfloat16).reshape(C, BLK)
    acc = (
        jnp.dot(ahi_ref[...], xt, preferred_element_type=jnp.float32)
        + jnp.dot(alo_ref[...], xt, preferred_element_type=jnp.float32)
        + c_ref[...]
    )
    o_ref[...] = acc.reshape(1, C, BLK // W, W)


def _fold_compute(w_ref, b_ref, ahi_ref, alo_ref, c_ref):
    wb = [w_ref[l].astype(jnp.bfloat16) for l in range(L)]
    nodes = []
    for k in range(L // 2):
        a = _dot(wb[2 * k + 1], wb[2 * k])
        c = _dot(wb[2 * k + 1], b_ref[2 * k][:, None]) + b_ref[2 * k + 1][:, None]
        nodes.append((a, c))
    while len(nodes) > 1:
        nxt = []
        for k in range(0, len(nodes) - 1, 2):
            (al, cl), (ar, cr) = nodes[k], nodes[k + 1]
            alh, all_ = _split(al)
            arh, arl = _split(ar)
            a = _dot(arh, alh) + _dot(arh, all_) + _dot(arl, alh)
            c = _dot(arh, cl.astype(jnp.bfloat16)) + cr
            nxt.append((a, c))
        if len(nodes) % 2:
            nxt.append(nodes[-1])
        nodes = nxt
    a, c = nodes[0]
    ahi = a.astype(jnp.bfloat16)
    ahi_ref[...] = ahi
    alo_ref[...] = (a - ahi.astype(jnp.float32)).astype(jnp.bfloat16)
    c_ref[...] = c


def kernel(x, W0, b0, W1, b1):
    out = pl.pallas_call(
        _fused_body,
        grid=(P // BLK,),
        in_specs=[
            pl.BlockSpec((1, C, BLK // W, W), lambda i: (0, 0, i, 0)),
            pl.BlockSpec((L, C, C), lambda i: (0, 0, 0)),
            pl.BlockSpec((L, C), lambda i: (0, 0)),
        ],
        out_specs=pl.BlockSpec((1, C, BLK // W, W), lambda i: (0, 0, i, 0)),
        out_shape=jax.ShapeDtypeStruct((1, C, H, W), jnp.float32),
        scratch_shapes=[
            pltpu.VMEM((C, C), jnp.bfloat16),
            pltpu.VMEM((C, C), jnp.bfloat16),
            pltpu.VMEM((C, 1), jnp.float32),
        ],
    )(x, W0, b0)
    return out
